# initial kernel scaffold (unmeasured)
import jax
import jax.numpy as jnp
from jax import lax
from jax.experimental import pallas as pl
from jax.experimental.pallas import tpu as pltpu

N_DEV = 32
B, SQ, DM = 2, 512, 768
HL, DH = 8, 64
HD = HL * DH
ROWS = B * SQ
CHUNK = ROWS // N_DEV
BLK = 64


def kernel(x, Wq, K_ext, V_ext, Wo):
    me_out = lax.axis_index("i")
    wq_l = lax.dynamic_slice(Wq, (0, me_out * HD), (DM, HD))
    wo_l = lax.dynamic_slice(Wo, (me_out * HD, 0), (HD, DM))

    def body(x_ref, wq_ref, k_ref, v_ref, wo_ref, out_ref,
             acc_ref, ctx_ref, rs_buf,
             rs_send, rs_recv, ag_send, ag_recv):
        me = lax.axis_index("i")

        qb = lax.broadcasted_iota(jnp.int32, (SQ, SQ), 0) // BLK
        kb = lax.broadcasted_iota(jnp.int32, (SQ, SQ), 1) // BLK
        mask = kb <= qb
        for b in range(B):
            q = jnp.dot(x_ref[b], wq_ref[...],
                        preferred_element_type=jnp.float32)
            for h in range(HL):
                qh = q[:, h * DH:(h + 1) * DH]
                kh = k_ref[b, :, h, :]
                vh = v_ref[b, :, h, :]
                s = lax.dot_general(qh, kh, (((1,), (1,)), ((), ())),
                                    preferred_element_type=jnp.float32)
                s = jnp.where(mask, s * 0.125, -1e9)
                m = jnp.max(s, axis=1, keepdims=True)
                e = jnp.exp(s - m)
                w = e / jnp.sum(e, axis=1, keepdims=True)
                ctx_ref[:, h * DH:(h + 1) * DH] = jnp.dot(
                    w, vh, preferred_element_type=jnp.float32)
            acc_ref[pl.ds(b * SQ, SQ), :] = jnp.dot(
                ctx_ref[...], wo_ref[...],
                preferred_element_type=jnp.float32)

        rs_descs = []
        for d in range(1, N_DEV):
            tgt = lax.rem(me + d, N_DEV)
            desc = pltpu.make_async_remote_copy(
                src_ref=acc_ref.at[pl.ds(tgt * CHUNK, CHUNK)],
                dst_ref=rs_buf.at[d - 1],
                send_sem=rs_send.at[d - 1],
                recv_sem=rs_recv.at[d - 1],
                device_id=(tgt,),
                device_id_type=pl.DeviceIdType.MESH,
            )
            desc.start()
            rs_descs.append(desc)

        for desc in rs_descs:
            desc.wait_recv()
        my_off = me * CHUNK
        chunk = acc_ref[pl.ds(my_off, CHUNK), :]
        for d in range(1, N_DEV):
            chunk = chunk + rs_buf[d - 1]
        acc_ref[pl.ds(my_off, CHUNK), :] = chunk

        ag_descs = []
        for d in range(1, N_DEV):
            tgt = lax.rem(me + d, N_DEV)
            desc = pltpu.make_async_remote_copy(
                src_ref=acc_ref.at[pl.ds(my_off, CHUNK)],
                dst_ref=acc_ref.at[pl.ds(my_off, CHUNK)],
                send_sem=ag_send.at[d - 1],
                recv_sem=ag_recv.at[d - 1],
                device_id=(tgt,),
                device_id_type=pl.DeviceIdType.MESH,
            )
            desc.start()
            ag_descs.append(desc)
        for desc in ag_descs:
            desc.wait_recv()

        out_ref[0, :, :] = acc_ref[pl.ds(0, SQ), :]
        out_ref[1, :, :] = acc_ref[pl.ds(SQ, SQ), :]

        for desc in rs_descs:
            desc.wait_send()
        for desc in ag_descs:
            desc.wait_send()

    return pl.pallas_call(
        body,
        out_shape=jax.ShapeDtypeStruct((B, SQ, DM), jnp.float32),
        in_specs=[pl.BlockSpec(memory_space=pltpu.VMEM)] * 5,
        out_specs=pl.BlockSpec(memory_space=pltpu.VMEM),
        scratch_shapes=[
            pltpu.VMEM((ROWS, DM), jnp.float32),
            pltpu.VMEM((SQ, HD), jnp.float32),
            pltpu.VMEM((N_DEV - 1, CHUNK, DM), jnp.float32),
            pltpu.SemaphoreType.DMA((N_DEV - 1,)),
            pltpu.SemaphoreType.DMA((N_DEV - 1,)),
            pltpu.SemaphoreType.DMA((N_DEV - 1,)),
            pltpu.SemaphoreType.DMA((N_DEV - 1,)),
        ],
        compiler_params=pltpu.CompilerParams(collective_id=0),
    )(x, wq_l, K_ext, V_ext, wo_l)


# baseline (device time: 114308 ns/iter reference)
import jax
import jax.numpy as jnp
from jax import lax
from jax.experimental import pallas as pl
from jax.experimental.pallas import tpu as pltpu

N_DEV = 32
B, SQ, DM = 2, 512, 768
HL, DH = 8, 64
HD = HL * DH
ROWS = B * SQ
CHUNK = ROWS // N_DEV
BLK = 64


def kernel(x, Wq, K_ext, V_ext, Wo):
    me_out = lax.axis_index("i")
    wq_l = lax.dynamic_slice(Wq, (0, me_out * HD), (DM, HD))
    wo_l = lax.dynamic_slice(Wo, (me_out * HD, 0), (HD, DM))

    def body(x_ref, wq_ref, k_ref, v_ref, wo_ref, out_ref,
             acc_ref, ctx_ref, rs_buf,
             rs_send, rs_recv, ag_send, ag_recv):
        me = lax.axis_index("i")

        qb = lax.broadcasted_iota(jnp.int32, (SQ, SQ), 0) // BLK
        kb = lax.broadcasted_iota(jnp.int32, (SQ, SQ), 1) // BLK
        mask = kb <= qb
        for b in range(B):
            q = jnp.dot(x_ref[b], wq_ref[...],
                        preferred_element_type=jnp.float32)
            for h in range(HL):
                qh = q[:, h * DH:(h + 1) * DH]
                kh = k_ref[b, :, h, :]
                vh = v_ref[b, :, h, :]
                s = lax.dot_general(qh, kh, (((1,), (1,)), ((), ())),
                                    preferred_element_type=jnp.float32)
                s = jnp.where(mask, s * 0.125, -1e9)
                m = jnp.max(s, axis=1, keepdims=True)
                e = jnp.exp(s - m)
                w = e / jnp.sum(e, axis=1, keepdims=True)
                ctx_ref[:, h * DH:(h + 1) * DH] = jnp.dot(
                    w, vh, preferred_element_type=jnp.float32)
            acc_ref[pl.ds(b * SQ, SQ), :] = jnp.dot(
                ctx_ref[...], wo_ref[...],
                preferred_element_type=jnp.float32)

        rs_descs = []
        for d in range(1, N_DEV):
            tgt = lax.rem(me + d, N_DEV)
            desc = pltpu.make_async_remote_copy(
                src_ref=acc_ref.at[pl.ds(tgt * CHUNK, CHUNK)],
                dst_ref=rs_buf.at[d - 1],
                send_sem=rs_send.at[d - 1],
                recv_sem=rs_recv.at[d - 1],
                device_id=(tgt,),
                device_id_type=pl.DeviceIdType.MESH,
            )
            desc.start()
            rs_descs.append(desc)

        for desc in rs_descs:
            desc.wait_recv()
        my_off = me * CHUNK
        chunk = acc_ref[pl.ds(my_off, CHUNK), :]
        for d in range(1, N_DEV):
            chunk = chunk + rs_buf[d - 1]
        acc_ref[pl.ds(my_off, CHUNK), :] = chunk

        ag_descs = []
        for d in range(1, N_DEV):
            tgt = lax.rem(me + d, N_DEV)
            desc = pltpu.make_async_remote_copy(
                src_ref=acc_ref.at[pl.ds(my_off, CHUNK)],
                dst_ref=acc_ref.at[pl.ds(my_off, CHUNK)],
                send_sem=ag_send.at[d - 1],
                recv_sem=ag_recv.at[d - 1],
                device_id=(tgt,),
                device_id_type=pl.DeviceIdType.MESH,
            )
            desc.start()
            ag_descs.append(desc)
        for desc in ag_descs:
            desc.wait_recv()

        out_ref[0, :, :] = acc_ref[pl.ds(0, SQ), :]
        out_ref[1, :, :] = acc_ref[pl.ds(SQ, SQ), :]

        for desc in rs_descs:
            desc.wait_send()
        for desc in ag_descs:
            desc.wait_send()

    return pl.pallas_call(
        body,
        out_shape=jax.ShapeDtypeStruct((B, SQ, DM), jnp.float32),
        in_specs=[pl.BlockSpec(memory_space=pltpu.VMEM)] * 5,
        out_specs=pl.BlockSpec(memory_space=pltpu.VMEM),
        scratch_shapes=[
            pltpu.VMEM((ROWS, DM), jnp.float32),
            pltpu.VMEM((SQ, HD), jnp.float32),
            pltpu.VMEM((N_DEV - 1, CHUNK, DM), jnp.float32),
            pltpu.SemaphoreType.DMA((N_DEV - 1,)),
            pltpu.SemaphoreType.DMA((N_DEV - 1,)),
            pltpu.SemaphoreType.DMA((N_DEV - 1,)),
            pltpu.SemaphoreType.DMA((N_DEV - 1,)),
        ],
    )(x, wq_l, K_ext, V_ext, wo_l)


# device time: 86487 ns/iter; 1.3217x vs baseline; 1.3217x over previous
import jax
import jax.numpy as jnp
from jax import lax
from jax.experimental import pallas as pl
from jax.experimental.pallas import tpu as pltpu

N_DEV = 32
B, SQ, DM = 2, 512, 768
HL, DH = 8, 64
HD = HL * DH
ROWS = B * SQ
CHUNK = ROWS // N_DEV
CPB = SQ // CHUNK
BLK = 64
BF = jnp.bfloat16
F32 = jnp.float32


def kernel(x, Wq, K_ext, V_ext, Wo):
    me_out = lax.axis_index("i")
    wq_l = lax.dynamic_slice(Wq, (0, me_out * HD), (DM, HD)).astype(BF)
    wo_l = lax.dynamic_slice(Wo, (me_out * HD, 0), (HD, DM)).astype(BF)
    x16 = x.astype(BF)
    k16 = K_ext.astype(BF)
    v16 = V_ext.astype(BF)

    def body(x_ref, wq_ref, k_ref, v_ref, wo_ref, out_ref,
             acc_ref, ctx_ref, rs_buf, g_buf,
             rs_send, rs_recv, ag_send, ag_recv):
        me = lax.axis_index("i")
        my_off = me * CHUNK

        rs_descs = []
        for d in range(1, N_DEV):
            tgt = lax.rem(me + d, N_DEV)
            rs_descs.append((tgt, pltpu.make_async_remote_copy(
                src_ref=acc_ref.at[pl.ds(tgt * CHUNK, CHUNK)],
                dst_ref=rs_buf.at[d - 1],
                send_sem=rs_send.at[d - 1],
                recv_sem=rs_recv.at[d - 1],
                device_id=(tgt,),
                device_id_type=pl.DeviceIdType.MESH,
            )))

        qb = lax.broadcasted_iota(jnp.int32, (SQ, SQ), 0) // BLK
        kb = lax.broadcasted_iota(jnp.int32, (SQ, SQ), 1) // BLK
        mask = kb <= qb
        for b in range(B):
            q = jnp.dot(x_ref[b], wq_ref[...],
                        preferred_element_type=F32).astype(BF)
            for h in range(HL):
                qh = q[:, h * DH:(h + 1) * DH]
                kh = k_ref[b, :, h, :]
                vh = v_ref[b, :, h, :]
                s = lax.dot_general(qh, kh, (((1,), (1,)), ((), ())),
                                    preferred_element_type=F32)
                s = jnp.where(mask, s * 0.125, -1e9)
                m = jnp.max(s, axis=1, keepdims=True)
                e = jnp.exp(s - m)
                w = (e / jnp.sum(e, axis=1, keepdims=True)).astype(BF)
                ctx_ref[:, h * DH:(h + 1) * DH] = jnp.dot(
                    w, vh, preferred_element_type=F32).astype(BF)
            acc_ref[pl.ds(b * SQ, SQ), :] = jnp.dot(
                ctx_ref[...], wo_ref[...],
                preferred_element_type=F32).astype(BF)
            for tgt, desc in rs_descs:
                in_batch = jnp.logical_and(tgt >= b * CPB, tgt < (b + 1) * CPB)
                pl.when(in_batch)(desc.start)

        for _, desc in rs_descs:
            desc.wait_recv()
        chunk = acc_ref[pl.ds(my_off, CHUNK), :].astype(F32)
        for d in range(1, N_DEV):
            chunk = chunk + rs_buf[d - 1].astype(F32)
        g_buf[pl.ds(my_off, CHUNK), :] = chunk.astype(BF)

        ag_descs = []
        for d in range(1, N_DEV):
            tgt = lax.rem(me + d, N_DEV)
            desc = pltpu.make_async_remote_copy(
                src_ref=g_buf.at[pl.ds(my_off, CHUNK)],
                dst_ref=g_buf.at[pl.ds(my_off, CHUNK)],
                send_sem=ag_send.at[d - 1],
                recv_sem=ag_recv.at[d - 1],
                device_id=(tgt,),
                device_id_type=pl.DeviceIdType.MESH,
            )
            desc.start()
            ag_descs.append(desc)
        for desc in ag_descs:
            desc.wait_recv()

        out_ref[...] = g_buf[...].astype(F32)

        for _, desc in rs_descs:
            desc.wait_send()
        for desc in ag_descs:
            desc.wait_send()

    out = pl.pallas_call(
        body,
        out_shape=jax.ShapeDtypeStruct((ROWS, DM), F32),
        in_specs=[pl.BlockSpec(memory_space=pltpu.VMEM)] * 5,
        out_specs=pl.BlockSpec(memory_space=pltpu.VMEM),
        scratch_shapes=[
            pltpu.VMEM((ROWS, DM), BF),
            pltpu.VMEM((SQ, HD), BF),
            pltpu.VMEM((N_DEV - 1, CHUNK, DM), BF),
            pltpu.VMEM((ROWS, DM), BF),
            pltpu.SemaphoreType.DMA((N_DEV - 1,)),
            pltpu.SemaphoreType.DMA((N_DEV - 1,)),
            pltpu.SemaphoreType.DMA((N_DEV - 1,)),
            pltpu.SemaphoreType.DMA((N_DEV - 1,)),
        ],
    )(x16, wq_l, k16, v16, wo_l)
    return out.reshape(B, SQ, DM)


# device time: 71304 ns/iter; 1.6031x vs baseline; 1.2129x over previous
import jax
import jax.numpy as jnp
from jax import lax
from jax.experimental import pallas as pl
from jax.experimental.pallas import tpu as pltpu

N_DEV = 32
NG = N_DEV // 2
B, SQ, DM = 2, 512, 768
HL, DH = 8, 64
HD = HL * DH
ROWS = B * SQ
CHUNK = SQ // NG
BLK = 64
BF = jnp.bfloat16
F32 = jnp.float32


def kernel(x, Wq, K_ext, V_ext, Wo):
    me_out = lax.axis_index("i")
    wq_l = lax.dynamic_slice(Wq, (0, me_out * HD), (DM, HD)).astype(BF)
    wo_l = lax.dynamic_slice(Wo, (me_out * HD, 0), (HD, DM)).astype(BF)
    x16 = x.astype(BF)
    k16 = K_ext.astype(BF)
    v16 = V_ext.astype(BF)

    def body(x_ref, wq_ref, k_ref, v_ref, wo_ref, out_ref,
             acc_ref, ctx_ref, p1_buf, rs_buf, gh_buf, oh_buf,
             s1_send, s1_recv, rs_send, rs_recv,
             ag_send, ag_recv, s4_send, s4_recv):
        me = lax.axis_index("i")
        parity = lax.rem(me, 2)
        partner = me + 1 - 2 * parity
        g = me // 2
        half_off = parity * SQ
        my_chunk = half_off + g * CHUNK

        s1 = pltpu.make_async_remote_copy(
            src_ref=acc_ref.at[pl.ds((1 - parity) * SQ, SQ)],
            dst_ref=p1_buf,
            send_sem=s1_send, recv_sem=s1_recv,
            device_id=(partner,), device_id_type=pl.DeviceIdType.MESH,
        )

        qb = lax.broadcasted_iota(jnp.int32, (SQ, SQ), 0) // BLK
        kb = lax.broadcasted_iota(jnp.int32, (SQ, SQ), 1) // BLK
        mask = kb <= qb
        for phase in range(B):
            b = (1 - parity) if phase == 0 else parity
            q = jnp.dot(x_ref[b], wq_ref[...],
                        preferred_element_type=F32).astype(BF)
            for h in range(HL):
                qh = q[:, h * DH:(h + 1) * DH]
                kh = k_ref[b, :, h, :]
                vh = v_ref[b, :, h, :]
                s = lax.dot_general(qh, kh, (((1,), (1,)), ((), ())),
                                    preferred_element_type=F32)
                s = jnp.where(mask, s * 0.125, -1e9)
                m = jnp.max(s, axis=1, keepdims=True)
                e = jnp.exp(s - m)
                w = (e / jnp.sum(e, axis=1, keepdims=True)).astype(BF)
                ctx_ref[:, h * DH:(h + 1) * DH] = jnp.dot(
                    w, vh, preferred_element_type=F32).astype(BF)
            acc_ref[pl.ds(b * SQ, SQ), :] = jnp.dot(
                ctx_ref[...], wo_ref[...],
                preferred_element_type=F32).astype(BF)
            if phase == 0:
                s1.start()

        s1.wait_recv()
        acc_ref[pl.ds(half_off, SQ), :] = (
            acc_ref[pl.ds(half_off, SQ), :].astype(F32)
            + p1_buf[...].astype(F32)).astype(BF)

        rs_descs = []
        for dg in range(1, NG):
            tg = lax.rem(g + dg, NG)
            desc = pltpu.make_async_remote_copy(
                src_ref=acc_ref.at[pl.ds(half_off + tg * CHUNK, CHUNK)],
                dst_ref=rs_buf.at[dg - 1],
                send_sem=rs_send.at[dg - 1],
                recv_sem=rs_recv.at[dg - 1],
                device_id=(2 * tg + parity,),
                device_id_type=pl.DeviceIdType.MESH,
            )
            desc.start()
            rs_descs.append(desc)
        for desc in rs_descs:
            desc.wait_recv()
        chunk = acc_ref[pl.ds(my_chunk, CHUNK), :].astype(F32)
        for dg in range(1, NG):
            chunk = chunk + rs_buf[dg - 1].astype(F32)
        gh_buf[pl.ds(g * CHUNK, CHUNK), :] = chunk.astype(BF)

        ag_descs = []
        for dg in range(1, NG):
            tg = lax.rem(g + dg, NG)
            desc = pltpu.make_async_remote_copy(
                src_ref=gh_buf.at[pl.ds(g * CHUNK, CHUNK)],
                dst_ref=gh_buf.at[pl.ds(g * CHUNK, CHUNK)],
                send_sem=ag_send.at[dg - 1],
                recv_sem=ag_recv.at[dg - 1],
                device_id=(2 * tg + parity,),
                device_id_type=pl.DeviceIdType.MESH,
            )
            desc.start()
            ag_descs.append(desc)
        for desc in ag_descs:
            desc.wait_recv()

        s4 = pltpu.make_async_remote_copy(
            src_ref=gh_buf, dst_ref=oh_buf,
            send_sem=s4_send, recv_sem=s4_recv,
            device_id=(partner,), device_id_type=pl.DeviceIdType.MESH,
        )
        s4.start()
        out_ref[pl.ds(half_off, SQ), :] = gh_buf[...].astype(F32)
        s4.wait_recv()
        out_ref[pl.ds((1 - parity) * SQ, SQ), :] = oh_buf[...].astype(F32)

        s1.wait_send()
        for desc in rs_descs:
            desc.wait_send()
        for desc in ag_descs:
            desc.wait_send()
        s4.wait_send()

    out = pl.pallas_call(
        body,
        out_shape=jax.ShapeDtypeStruct((ROWS, DM), F32),
        in_specs=[pl.BlockSpec(memory_space=pltpu.VMEM)] * 5,
        out_specs=pl.BlockSpec(memory_space=pltpu.VMEM),
        scratch_shapes=[
            pltpu.VMEM((ROWS, DM), BF),
            pltpu.VMEM((SQ, HD), BF),
            pltpu.VMEM((SQ, DM), BF),
            pltpu.VMEM((NG - 1, CHUNK, DM), BF),
            pltpu.VMEM((SQ, DM), BF),
            pltpu.VMEM((SQ, DM), BF),
            pltpu.SemaphoreType.DMA,
            pltpu.SemaphoreType.DMA,
            pltpu.SemaphoreType.DMA((NG - 1,)),
            pltpu.SemaphoreType.DMA((NG - 1,)),
            pltpu.SemaphoreType.DMA((NG - 1,)),
            pltpu.SemaphoreType.DMA((NG - 1,)),
            pltpu.SemaphoreType.DMA,
            pltpu.SemaphoreType.DMA,
        ],
    )(x16, wq_l, k16, v16, wo_l)
    return out.reshape(B, SQ, DM)


# device time: 67504 ns/iter; 1.6934x vs baseline; 1.0563x over previous
import jax
import jax.numpy as jnp
from jax import lax
from jax.experimental import pallas as pl
from jax.experimental.pallas import tpu as pltpu

N_DEV = 32
NG = N_DEV // 2
B, SQ, DM = 2, 512, 768
HL, DH = 8, 64
HD = HL * DH
ROWS = B * SQ
CHUNK = SQ // NG
BLK = 64
BF = jnp.bfloat16
F32 = jnp.float32


def kernel(x, Wq, K_ext, V_ext, Wo):
    me_out = lax.axis_index("i")
    wq_l = lax.dynamic_slice(Wq, (0, me_out * HD), (DM, HD)).astype(BF)
    wo_l = lax.dynamic_slice(Wo, (me_out * HD, 0), (HD, DM)).astype(BF)
    x16 = x.astype(BF)
    k16 = K_ext.astype(BF)
    v16 = V_ext.astype(BF)

    def body(x_ref, wq_ref, k_ref, v_ref, wo_ref, out_ref,
             acc_ref, ctx_ref, p1_buf, rs_buf, gh_buf, oh_buf,
             s1_send, s1_recv, rs_send, rs_recv,
             ag_send, ag_recv, f_send, f_recv):
        me = lax.axis_index("i")
        parity = lax.rem(me, 2)
        partner = me + 1 - 2 * parity
        g = me // 2
        half_off = parity * SQ
        my_chunk = half_off + g * CHUNK

        s1 = pltpu.make_async_remote_copy(
            src_ref=acc_ref.at[pl.ds((1 - parity) * SQ, SQ)],
            dst_ref=p1_buf,
            send_sem=s1_send, recv_sem=s1_recv,
            device_id=(partner,), device_id_type=pl.DeviceIdType.MESH,
        )

        qb = lax.broadcasted_iota(jnp.int32, (SQ, SQ), 0) // BLK
        kb = lax.broadcasted_iota(jnp.int32, (SQ, SQ), 1) // BLK
        mask = kb <= qb
        for phase in range(B):
            b = (1 - parity) if phase == 0 else parity
            q = jnp.dot(x_ref[b], wq_ref[...],
                        preferred_element_type=F32).astype(BF)
            for h in range(HL):
                qh = q[:, h * DH:(h + 1) * DH]
                kh = k_ref[b, :, h, :]
                vh = v_ref[b, :, h, :]
                s = lax.dot_general(qh, kh, (((1,), (1,)), ((), ())),
                                    preferred_element_type=F32)
                e = jnp.exp(jnp.where(mask, s * 0.125, -1e9))
                w = (e / jnp.sum(e, axis=1, keepdims=True)).astype(BF)
                ctx_ref[:, h * DH:(h + 1) * DH] = jnp.dot(
                    w, vh, preferred_element_type=F32).astype(BF)
            acc_ref[pl.ds(b * SQ, SQ), :] = jnp.dot(
                ctx_ref[...], wo_ref[...],
                preferred_element_type=F32).astype(BF)
            if phase == 0:
                s1.start()

        s1.wait_recv()
        acc_ref[pl.ds(half_off, SQ), :] = (
            acc_ref[pl.ds(half_off, SQ), :].astype(F32)
            + p1_buf[...].astype(F32)).astype(BF)

        rs_descs = []
        for dg in range(1, NG):
            tg = lax.rem(g + dg, NG)
            desc = pltpu.make_async_remote_copy(
                src_ref=acc_ref.at[pl.ds(half_off + tg * CHUNK, CHUNK)],
                dst_ref=rs_buf.at[dg - 1],
                send_sem=rs_send.at[dg - 1],
                recv_sem=rs_recv.at[dg - 1],
                device_id=(2 * tg + parity,),
                device_id_type=pl.DeviceIdType.MESH,
            )
            desc.start()
            rs_descs.append(desc)
        for desc in rs_descs:
            desc.wait_recv()
        chunk = acc_ref[pl.ds(my_chunk, CHUNK), :].astype(F32)
        for dg in range(1, NG):
            chunk = chunk + rs_buf[dg - 1].astype(F32)
        gh_buf[pl.ds(g * CHUNK, CHUNK), :] = chunk.astype(BF)

        ag_descs = []
        for dg in range(1, NG):
            tg = lax.rem(g + dg, NG)
            desc = pltpu.make_async_remote_copy(
                src_ref=gh_buf.at[pl.ds(g * CHUNK, CHUNK)],
                dst_ref=gh_buf.at[pl.ds(g * CHUNK, CHUNK)],
                send_sem=ag_send.at[dg - 1],
                recv_sem=ag_recv.at[dg - 1],
                device_id=(2 * tg + parity,),
                device_id_type=pl.DeviceIdType.MESH,
            )
            desc.start()
            ag_descs.append(desc)

        f_descs = []
        for dg in range(NG):
            r = lax.rem(g - dg + NG, NG) * CHUNK
            f_descs.append(pltpu.make_async_remote_copy(
                src_ref=gh_buf.at[pl.ds(r, CHUNK)],
                dst_ref=oh_buf.at[pl.ds(r, CHUNK)],
                send_sem=f_send.at[dg],
                recv_sem=f_recv.at[dg],
                device_id=(partner,), device_id_type=pl.DeviceIdType.MESH,
            ))
        f_descs[0].start()
        for dg in range(1, NG):
            ag_descs[dg - 1].wait_recv()
            f_descs[dg].start()

        out_ref[pl.ds(half_off, SQ), :] = gh_buf[...].astype(F32)
        for desc in f_descs:
            desc.wait_recv()
        out_ref[pl.ds((1 - parity) * SQ, SQ), :] = oh_buf[...].astype(F32)

        s1.wait_send()
        for desc in rs_descs:
            desc.wait_send()
        for desc in ag_descs:
            desc.wait_send()
        for desc in f_descs:
            desc.wait_send()

    out = pl.pallas_call(
        body,
        out_shape=jax.ShapeDtypeStruct((ROWS, DM), F32),
        in_specs=[pl.BlockSpec(memory_space=pltpu.VMEM)] * 5,
        out_specs=pl.BlockSpec(memory_space=pltpu.VMEM),
        scratch_shapes=[
            pltpu.VMEM((ROWS, DM), BF),
            pltpu.VMEM((SQ, HD), BF),
            pltpu.VMEM((SQ, DM), BF),
            pltpu.VMEM((NG - 1, CHUNK, DM), BF),
            pltpu.VMEM((SQ, DM), BF),
            pltpu.VMEM((SQ, DM), BF),
            pltpu.SemaphoreType.DMA,
            pltpu.SemaphoreType.DMA,
            pltpu.SemaphoreType.DMA((NG - 1,)),
            pltpu.SemaphoreType.DMA((NG - 1,)),
            pltpu.SemaphoreType.DMA((NG - 1,)),
            pltpu.SemaphoreType.DMA((NG - 1,)),
            pltpu.SemaphoreType.DMA((NG,)),
            pltpu.SemaphoreType.DMA((NG,)),
        ],
    )(x16, wq_l, k16, v16, wo_l)
    return out.reshape(B, SQ, DM)
